# Initial kernel scaffold; baseline (speedup 1.0000x reference)
import jax
import jax.numpy as jnp
from jax.experimental import pallas as pl


def kernel(h, edge_index, W_self, W_neigh, b, index):
    raise NotImplementedError("write your pallas kernel here")


# speedup vs baseline: 5.6174x; 5.6174x over previous
import functools

import jax
import jax.numpy as jnp
from jax import lax
from jax.experimental import pallas as pl
from jax.experimental.pallas import tpu as pltpu
from jax.experimental.pallas import tpu_sc as plsc

N_NODES = 10000
D = 128

NC = 2
NS = 16
B = 128
CHUNKS = 79
E_PAD = NC * NS * CHUNKS * B
ROWS_PER_TILE = 632
ACC_ROWS = NS * ROWS_PER_TILE
OUT_ROWS_PER_TILE = 624
OUT_ROWS_LAST = N_NODES - 15 * OUT_ROWS_PER_TILE


def _sc_segment_sum(src_r, dst_r, h, zeros_chunk):
  mesh = plsc.VectorSubcoreMesh(core_axis_name="c", subcore_axis_name="s")

  @functools.partial(
      pl.kernel,
      mesh=mesh,
      out_type=jax.ShapeDtypeStruct((NC, N_NODES, D), jnp.float32),
      scratch_types=[
          pltpu.VMEM((CHUNKS, B), jnp.int32),
          pltpu.VMEM((CHUNKS, B), jnp.int32),
          pltpu.VMEM((B, D), jnp.float32),
          pltpu.VMEM_SHARED((ACC_ROWS, D), jnp.float32),
          pltpu.SemaphoreType.DMA,
      ],
  )
  def seg_sum(src_hbm, dst_hbm, h_hbm, z_hbm, out_hbm,
              src_v, dst_v, rows_v, acc_sh, sem):
    c = lax.axis_index("c")
    s = lax.axis_index("s")

    pltpu.sync_copy(z_hbm, acc_sh.at[pl.ds(s * ROWS_PER_TILE, ROWS_PER_TILE)])
    pltpu.sync_copy(src_hbm.at[c, s], src_v)
    pltpu.sync_copy(dst_hbm.at[c, s], dst_v)
    plsc.subcore_barrier()

    def body(j, carry):
      pltpu.async_copy(h_hbm.at[src_v.at[j]], rows_v, sem).wait()
      pltpu.sync_copy(rows_v, acc_sh.at[dst_v.at[j]], add=True)
      return carry

    lax.fori_loop(0, CHUNKS, body, 0)
    plsc.subcore_barrier()

    base = s * OUT_ROWS_PER_TILE

    @pl.when(s < NS - 1)
    def _():
      pltpu.sync_copy(acc_sh.at[pl.ds(base, OUT_ROWS_PER_TILE)],
                      out_hbm.at[c, pl.ds(base, OUT_ROWS_PER_TILE)])

    @pl.when(s == NS - 1)
    def _():
      last = (NS - 1) * OUT_ROWS_PER_TILE
      pltpu.sync_copy(acc_sh.at[pl.ds(last, OUT_ROWS_LAST)],
                      out_hbm.at[c, pl.ds(last, OUT_ROWS_LAST)])

  return seg_sum(src_r, dst_r, h, zeros_chunk)


def _tc_combine_body(h_ref, p_ref, ws_ref, wn_ref, b_ref, o_ref):
  agg = p_ref[0] + p_ref[1]
  acc = jnp.dot(h_ref[...], ws_ref[...], preferred_element_type=jnp.float32)
  acc = acc + jnp.dot(agg, wn_ref[...], preferred_element_type=jnp.float32)
  o_ref[...] = jnp.maximum(acc + b_ref[...], 0.0)


def _tc_combine(h, partials, w_self, w_neigh, b):
  blk = 1000
  grid = (N_NODES // blk,)
  return pl.pallas_call(
      _tc_combine_body,
      grid=grid,
      in_specs=[
          pl.BlockSpec((blk, D), lambda i: (i, 0)),
          pl.BlockSpec((NC, blk, D), lambda i: (0, i, 0)),
          pl.BlockSpec((D, D), lambda i: (0, 0)),
          pl.BlockSpec((D, D), lambda i: (0, 0)),
          pl.BlockSpec((1, D), lambda i: (0, 0)),
      ],
      out_specs=pl.BlockSpec((blk, D), lambda i: (i, 0)),
      out_shape=jax.ShapeDtypeStruct((N_NODES, D), jnp.float32),
  )(h, partials, w_self, w_neigh, b.reshape(1, D))


def kernel(h, edge_index, W_self, W_neigh, b, index):
  del index
  src = edge_index[0].astype(jnp.int32)
  dst = edge_index[1].astype(jnp.int32)
  pad = E_PAD - src.shape[0]
  src_p = jnp.concatenate([src, jnp.zeros((pad,), jnp.int32)])
  dst_p = jnp.concatenate([dst, jnp.full((pad,), N_NODES, jnp.int32)])
  src_r = src_p.reshape(NC, NS, CHUNKS, B)
  dst_r = dst_p.reshape(NC, NS, CHUNKS, B)
  zeros_chunk = jnp.zeros((ROWS_PER_TILE, D), jnp.float32)
  partials = _sc_segment_sum(src_r, dst_r, h, zeros_chunk)
  return _tc_combine(h, partials, W_self, W_neigh, b)
